# Initial kernel scaffold; baseline (speedup 1.0000x reference)
import jax
import jax.numpy as jnp
from jax.experimental import pallas as pl


def kernel(x, edge_index, W1, a1_src, a1_dst, b1, W2, a2_src, a2_dst, b2, W3, a3_src, a3_dst, b3):
    raise NotImplementedError("write your pallas kernel here")


# speedup vs baseline: 26.3057x; 26.3057x over previous
import functools

import jax
import jax.numpy as jnp
from jax import lax
from jax.experimental import pallas as pl
from jax.experimental.pallas import tpu as pltpu
from jax.experimental.pallas import tpu_sc as plsc

N_NODES = 10000
NP = 10240
E = 320000
NW = 32
EPW = E // NW
K = 80
NCHUNK = EPW // K
RPT = NP // 16
F32 = jnp.float32


def _make_sc_agg():
  mesh = plsc.VectorSubcoreMesh(core_axis_name="c", subcore_axis_name="s")

  def body(h_hbm, as_hbm, ad_hbm, src_hbm, dst_hbm, out_hbm, den_hbm,
           as_v, ad_v, src_b, dst_b, ex_v, rows_v, den_v, idx_v,
           gsem, out_sp, den_sp):
    c = lax.axis_index("c")
    s = lax.axis_index("s")
    wid = c * 16 + s

    pltpu.sync_copy(as_hbm, as_v)
    pltpu.sync_copy(ad_hbm, ad_v)

    def zden(i, _):
      for r in range(8):
        den_v[i, pl.ds(r * 16, 16)] = jnp.zeros((16,), F32)
      return 0
    lax.fori_loop(0, NP // 128, zden, 0)

    def ziota(i, _):
      idx_v[pl.ds(i * 16, 16)] = lax.iota(jnp.int32, 16) + i * 16
      return 0
    lax.fori_loop(0, (NP // 128) // 16, ziota, 0)

    def zrow(i, _):
      for r in range(8):
        rows_v[i, pl.ds(r * 16, 16)] = jnp.zeros((16,), F32)
      return 0
    lax.fori_loop(0, K, zrow, 0)

    def zsp(i, _):
      pltpu.sync_copy(rows_v.at[pl.ds(0, K), :],
                      out_sp.at[pl.ds(s * RPT + i * K, K), :])
      return 0
    lax.fori_loop(0, RPT // K, zsp, 0)

    @pl.when(s < 10)
    def _():
      pltpu.sync_copy(den_v.at[pl.ds(s * 8, 8), :],
                      den_sp.at[pl.ds(s * 8, 8), :])

    def mx(i, m):
      return jnp.maximum(m, as_v[pl.ds(i * 16, 16)])
    m = lax.fori_loop(0, NP // 16, mx, jnp.full((16,), -1e30, F32))
    smax = jnp.float32(0.0)
    for lane in range(16):
      smax = jnp.maximum(smax, m[lane])

    plsc.subcore_barrier()

    ebase = wid * EPW

    def chunk(j, _):
      pltpu.sync_copy(src_hbm.at[pl.ds(ebase + j * K, K)], src_b.at[0])
      pltpu.sync_copy(dst_hbm.at[pl.ds(ebase + j * K, K)], dst_b.at[0])
      cp = pltpu.async_copy(h_hbm.at[src_b.at[0]], rows_v, gsem)
      for t in range(K // 16):
        srcv = src_b[0, pl.ds(t * 16, 16)]
        dstv = dst_b[0, pl.ds(t * 16, 16)]
        asg = plsc.load_gather(as_v, [srcv])
        adg = plsc.load_gather(ad_v, [dstv])
        t1 = asg + adg
        e = jnp.where(t1 >= 0, t1, 0.2 * t1)
        t2 = smax + adg
        cc = jnp.where(t2 >= 0, t2, 0.2 * t2)
        exv = jnp.exp(e - cc)
        ex_v[pl.ds(t * 16, 16)] = exv
        plsc.addupdate_scatter(den_v, [lax.shift_right_logical(dstv, 7),
                                       lax.bitwise_and(dstv, 127)], exv)
      cp.wait()

      def scale(t, _):
        av = ex_v[pl.ds(t * 16, 16)]
        for jj in range(16):
          a = av[jj]
          k = t * 16 + jj
          for r in range(8):
            rows_v[k, pl.ds(r * 16, 16)] = rows_v[k, pl.ds(r * 16, 16)] * a
        return 0
      lax.fori_loop(0, K // 16, scale, 0)

      pltpu.sync_copy(rows_v, out_sp.at[dst_b.at[0]], add=True)
      return 0
    lax.fori_loop(0, NCHUNK, chunk, 0)

    pltpu.sync_copy(den_v, den_sp.at[idx_v], add=True)
    plsc.subcore_barrier()

    def wb(i, _):
      base = s * RPT + i * K
      pltpu.sync_copy(out_sp.at[pl.ds(base, K), :], rows_v)
      pltpu.sync_copy(rows_v, out_hbm.at[c, pl.ds(base, K), :])
      return 0
    lax.fori_loop(0, RPT // K, wb, 0)

    @pl.when(s < 10)
    def _():
      pltpu.sync_copy(den_sp.at[pl.ds(s * 8, 8), :], den_v.at[pl.ds(0, 8), :])
      pltpu.sync_copy(den_v.at[pl.ds(0, 8), :],
                      den_hbm.at[c, pl.ds(s * 8, 8), :])

  return pl.kernel(
      body,
      out_type=(jax.ShapeDtypeStruct((2, NP, 128), F32),
                jax.ShapeDtypeStruct((2, NP // 128, 128), F32)),
      mesh=mesh,
      compiler_params=pltpu.CompilerParams(needs_layout_passes=False),
      scratch_types=[
          pltpu.VMEM((NP,), F32),
          pltpu.VMEM((NP,), F32),
          pltpu.VMEM((2, K), jnp.int32),
          pltpu.VMEM((2, K), jnp.int32),
          pltpu.VMEM((K,), F32),
          pltpu.VMEM((K, 128), F32),
          pltpu.VMEM((NP // 128, 128), F32),
          pltpu.VMEM((NP // 128,), jnp.int32),
          pltpu.SemaphoreType.DMA,
          pltpu.VMEM_SHARED((NP, 128), F32),
          pltpu.VMEM_SHARED((NP // 128, 128), F32),
      ],
  )


_DOT = functools.partial(lax.dot_general, preferred_element_type=F32)


def _tc_first(x, W, A2):
  def body(x_ref, w_ref, a2_ref, h_ref, st_ref):
    h = _DOT(x_ref[...], w_ref[...], (((1,), (0,)), ((), ())))
    h_ref[...] = h
    st_ref[...] = _DOT(a2_ref[...], h, (((1,), (1,)), ((), ())))

  return pl.pallas_call(
      body,
      grid=(NP // 1024,),
      in_specs=[pl.BlockSpec((1024, 128), lambda i: (i, 0)),
                pl.BlockSpec((128, 128), lambda i: (0, 0)),
                pl.BlockSpec((8, 128), lambda i: (0, 0))],
      out_specs=[pl.BlockSpec((1024, 128), lambda i: (i, 0)),
                 pl.BlockSpec((8, 1024), lambda i: (0, i))],
      out_shape=[jax.ShapeDtypeStruct((NP, 128), F32),
                 jax.ShapeDtypeStruct((8, NP), F32)],
  )(x, W, A2)


def _norm_block(op_ref, dp_ref):
  o3 = (op_ref[0] + op_ref[1]).reshape(8, 128, 128)
  inv = 1.0 / (dp_ref[0] + dp_ref[1] + 1e-16)
  eye = (lax.broadcasted_iota(jnp.int32, (128, 128), 0)
         == lax.broadcasted_iota(jnp.int32, (128, 128), 1)).astype(F32)
  diagm = eye[None] * inv[:, None, :]
  agg = _DOT(diagm, o3, (((2,), (1,)), ((0,), (0,))))
  return agg.reshape(1024, 128)


def _tc_mid(op, dp, bvec, W, A2):
  def body(op_ref, dp_ref, b_ref, w_ref, a2_ref, h_ref, st_ref):
    xin = jnp.maximum(_norm_block(op_ref, dp_ref) + b_ref[...], 0.0)
    h = _DOT(xin, w_ref[...], (((1,), (0,)), ((), ())))
    h_ref[...] = h
    st_ref[...] = _DOT(a2_ref[...], h, (((1,), (1,)), ((), ())))

  return pl.pallas_call(
      body,
      grid=(NP // 1024,),
      in_specs=[pl.BlockSpec((2, 1024, 128), lambda i: (0, i, 0)),
                pl.BlockSpec((2, 8, 128), lambda i: (0, i, 0)),
                pl.BlockSpec((1, 128), lambda i: (0, 0)),
                pl.BlockSpec((128, 128), lambda i: (0, 0)),
                pl.BlockSpec((8, 128), lambda i: (0, 0))],
      out_specs=[pl.BlockSpec((1024, 128), lambda i: (i, 0)),
                 pl.BlockSpec((8, 1024), lambda i: (0, i))],
      out_shape=[jax.ShapeDtypeStruct((NP, 128), F32),
                 jax.ShapeDtypeStruct((8, NP), F32)],
  )(op, dp, bvec, W, A2)


def _tc_final(op, dp, bvec):
  def body(op_ref, dp_ref, b_ref, out_ref):
    out_ref[...] = _norm_block(op_ref, dp_ref) + b_ref[...]

  return pl.pallas_call(
      body,
      grid=(NP // 1024,),
      in_specs=[pl.BlockSpec((2, 1024, 128), lambda i: (0, i, 0)),
                pl.BlockSpec((2, 8, 128), lambda i: (0, i, 0)),
                pl.BlockSpec((1, 128), lambda i: (0, 0))],
      out_specs=pl.BlockSpec((1024, 128), lambda i: (i, 0)),
      out_shape=jax.ShapeDtypeStruct((NP, 128), F32),
  )(op, dp, bvec)


def kernel(x, edge_index, W1, a1_src, a1_dst, b1,
           W2, a2_src, a2_dst, b2, W3, a3_src, a3_dst, b3):
  ei = edge_index.astype(jnp.int32)
  src = ei[0]
  dst = ei[1]

  xp = jnp.pad(x, ((0, NP - N_NODES), (0, 0)))

  def a2rows(a_s, a_d, d):
    z = jnp.zeros((8, d), F32)
    return z.at[0, :a_s.shape[0]].set(a_s).at[1, :a_d.shape[0]].set(a_d)

  sc = _make_sc_agg()

  h1, st1 = _tc_first(xp, W1, a2rows(a1_src, a1_dst, 128))
  op1, dp1 = sc(h1, st1[0], st1[1], src, dst)

  h2, st2 = _tc_mid(op1, dp1, b1.reshape(1, 128),
                    W2, a2rows(a2_src, a2_dst, 128))
  op2, dp2 = sc(h2, st2[0], st2[1], src, dst)

  W3p = jnp.pad(W3, ((0, 0), (0, 128 - W3.shape[1])))
  h3, st3 = _tc_mid(op2, dp2, b2.reshape(1, 128),
                    W3p, a2rows(a3_src, a3_dst, 128))
  op3, dp3 = sc(h3, st3[0], st3[1], src, dst)

  b3p = jnp.pad(b3, (0, 128 - b3.shape[0])).reshape(1, 128)
  out = _tc_final(op3, dp3, b3p)
  return out[:N_NODES, :40]


# speedup vs baseline: 38.0697x; 1.4472x over previous
import functools

import jax
import jax.numpy as jnp
from jax import lax
from jax.experimental import pallas as pl
from jax.experimental.pallas import tpu as pltpu
from jax.experimental.pallas import tpu_sc as plsc

N_NODES = 10000
NP = 10240
E = 320000
NW = 32
EPW = E // NW
K = 80
NCHUNK = EPW // K
RPT = NP // 16
F32 = jnp.float32


def _make_sc_agg():
  mesh = plsc.VectorSubcoreMesh(core_axis_name="c", subcore_axis_name="s")

  def body(h_hbm, as_hbm, ad_hbm, src_hbm, dst_hbm, out_hbm, den_hbm,
           as_v, ad_v, src_b, dst_b, ex_v, rows_v, den_v, idx_v,
           gsem, gsem_b, isem_s, idst_s, out_sp, den_sp):
    c = lax.axis_index("c")
    s = lax.axis_index("s")
    wid = c * 16 + s

    pltpu.sync_copy(as_hbm, as_v)
    pltpu.sync_copy(ad_hbm, ad_v)

    def zden(i, _):
      for r in range(8):
        den_v[i, pl.ds(r * 16, 16)] = jnp.zeros((16,), F32)
      return 0
    lax.fori_loop(0, NP // 128, zden, 0)

    def ziota(i, _):
      idx_v[pl.ds(i * 16, 16)] = lax.iota(jnp.int32, 16) + i * 16
      return 0
    lax.fori_loop(0, (NP // 128) // 16, ziota, 0)

    def zrow(i, _):
      for r in range(8):
        rows_v[i, pl.ds(r * 16, 16)] = jnp.zeros((16,), F32)
      return 0
    lax.fori_loop(0, K, zrow, 0)

    def zsp(i, _):
      pltpu.sync_copy(rows_v.at[pl.ds(0, K), :],
                      out_sp.at[pl.ds(s * RPT + i * K, K), :])
      return 0
    lax.fori_loop(0, RPT // K, zsp, 0)

    @pl.when(s < 10)
    def _():
      pltpu.sync_copy(den_v.at[pl.ds(s * 8, 8), :],
                      den_sp.at[pl.ds(s * 8, 8), :])

    def mx(i, m):
      return jnp.maximum(m, as_v[pl.ds(i * 16, 16)])
    m = lax.fori_loop(0, NP // 16, mx, jnp.full((16,), -1e30, F32))
    smax = jnp.float32(0.0)
    for lane in range(16):
      smax = jnp.maximum(smax, m[lane])

    plsc.subcore_barrier()

    ebase = wid * EPW
    KH = K // 2

    pltpu.async_copy(src_hbm.at[pl.ds(ebase, K)], src_b.at[0], isem_s)
    pltpu.async_copy(dst_hbm.at[pl.ds(ebase, K)], dst_b.at[0], idst_s)

    def chunk(j, _):
      p = lax.bitwise_and(j, 1)
      pn = 1 - p
      pltpu.make_async_copy(src_hbm.at[pl.ds(ebase, K)], src_b.at[p],
                            isem_s).wait()
      pltpu.make_async_copy(dst_hbm.at[pl.ds(ebase, K)], dst_b.at[p],
                            idst_s).wait()
      pltpu.async_copy(h_hbm.at[src_b.at[p, pl.ds(0, KH)]],
                       rows_v.at[pl.ds(0, KH), :], gsem)
      pltpu.async_copy(h_hbm.at[src_b.at[p, pl.ds(KH, KH)]],
                       rows_v.at[pl.ds(KH, KH), :], gsem_b)

      @pl.when(j + 1 < NCHUNK)
      def _():
        off = ebase + (j + 1) * K
        pltpu.async_copy(src_hbm.at[pl.ds(off, K)], src_b.at[pn], isem_s)
        pltpu.async_copy(dst_hbm.at[pl.ds(off, K)], dst_b.at[pn], idst_s)

      for t in range(K // 16):
        srcv = src_b[p, pl.ds(t * 16, 16)]
        dstv = dst_b[p, pl.ds(t * 16, 16)]
        asg = plsc.load_gather(as_v, [srcv])
        adg = plsc.load_gather(ad_v, [dstv])
        t1 = asg + adg
        e = jnp.where(t1 >= 0, t1, 0.2 * t1)
        t2 = smax + adg
        cc = jnp.where(t2 >= 0, t2, 0.2 * t2)
        exv = jnp.exp(e - cc)
        ex_v[pl.ds(t * 16, 16)] = exv
        plsc.addupdate_scatter(den_v, [lax.shift_right_logical(dstv, 7),
                                       lax.bitwise_and(dstv, 127)], exv)

      def scale(t, _):
        av = ex_v[pl.ds(t * 16, 16)]
        for jj in range(16):
          a = av[jj]
          k = t * 16 + jj
          for r in range(8):
            rows_v[k, pl.ds(r * 16, 16)] = rows_v[k, pl.ds(r * 16, 16)] * a
        return 0

      pltpu.make_async_copy(h_hbm.at[src_b.at[p, pl.ds(0, KH)]],
                            rows_v.at[pl.ds(0, KH), :], gsem).wait()
      lax.fori_loop(0, KH // 16, scale, 0)
      pltpu.make_async_copy(h_hbm.at[src_b.at[p, pl.ds(KH, KH)]],
                            rows_v.at[pl.ds(KH, KH), :], gsem_b).wait()
      lax.fori_loop(KH // 16, K // 16, scale, 0)

      pltpu.sync_copy(rows_v, out_sp.at[dst_b.at[p]], add=True)
      return 0
    lax.fori_loop(0, NCHUNK, chunk, 0)

    pltpu.sync_copy(den_v, den_sp.at[idx_v], add=True)
    plsc.subcore_barrier()

    def wb(i, _):
      base = s * RPT + i * K
      pltpu.sync_copy(out_sp.at[pl.ds(base, K), :], rows_v)
      pltpu.sync_copy(rows_v, out_hbm.at[c, pl.ds(base, K), :])
      return 0
    lax.fori_loop(0, RPT // K, wb, 0)

    @pl.when(s < 10)
    def _():
      pltpu.sync_copy(den_sp.at[pl.ds(s * 8, 8), :], den_v.at[pl.ds(0, 8), :])
      pltpu.sync_copy(den_v.at[pl.ds(0, 8), :],
                      den_hbm.at[c, pl.ds(s * 8, 8), :])

  return pl.kernel(
      body,
      out_type=(jax.ShapeDtypeStruct((2, NP, 128), F32),
                jax.ShapeDtypeStruct((2, NP // 128, 128), F32)),
      mesh=mesh,
      compiler_params=pltpu.CompilerParams(needs_layout_passes=False),
      scratch_types=[
          pltpu.VMEM((NP,), F32),
          pltpu.VMEM((NP,), F32),
          pltpu.VMEM((2, K), jnp.int32),
          pltpu.VMEM((2, K), jnp.int32),
          pltpu.VMEM((K,), F32),
          pltpu.VMEM((K, 128), F32),
          pltpu.VMEM((NP // 128, 128), F32),
          pltpu.VMEM((NP // 128,), jnp.int32),
          pltpu.SemaphoreType.DMA,
          pltpu.SemaphoreType.DMA,
          pltpu.SemaphoreType.DMA,
          pltpu.SemaphoreType.DMA,
          pltpu.VMEM_SHARED((NP, 128), F32),
          pltpu.VMEM_SHARED((NP // 128, 128), F32),
      ],
  )


_DOT = functools.partial(lax.dot_general, preferred_element_type=F32)


def _tc_first(x, W, A2):
  def body(x_ref, w_ref, a2_ref, h_ref, st_ref):
    h = _DOT(x_ref[...], w_ref[...], (((1,), (0,)), ((), ())))
    h_ref[...] = h
    st_ref[...] = _DOT(a2_ref[...], h, (((1,), (1,)), ((), ())))

  return pl.pallas_call(
      body,
      grid=(NP // 1024,),
      in_specs=[pl.BlockSpec((1024, 128), lambda i: (i, 0)),
                pl.BlockSpec((128, 128), lambda i: (0, 0)),
                pl.BlockSpec((8, 128), lambda i: (0, 0))],
      out_specs=[pl.BlockSpec((1024, 128), lambda i: (i, 0)),
                 pl.BlockSpec((8, 1024), lambda i: (0, i))],
      out_shape=[jax.ShapeDtypeStruct((NP, 128), F32),
                 jax.ShapeDtypeStruct((8, NP), F32)],
  )(x, W, A2)


def _norm_block(op_ref, dp_ref):
  o3 = (op_ref[0] + op_ref[1]).reshape(8, 128, 128)
  inv = 1.0 / (dp_ref[0] + dp_ref[1] + 1e-16)
  eye = (lax.broadcasted_iota(jnp.int32, (128, 128), 0)
         == lax.broadcasted_iota(jnp.int32, (128, 128), 1)).astype(F32)
  diagm = eye[None] * inv[:, None, :]
  agg = _DOT(diagm, o3, (((2,), (1,)), ((0,), (0,))))
  return agg.reshape(1024, 128)


def _tc_mid(op, dp, bvec, W, A2):
  def body(op_ref, dp_ref, b_ref, w_ref, a2_ref, h_ref, st_ref):
    xin = jnp.maximum(_norm_block(op_ref, dp_ref) + b_ref[...], 0.0)
    h = _DOT(xin, w_ref[...], (((1,), (0,)), ((), ())))
    h_ref[...] = h
    st_ref[...] = _DOT(a2_ref[...], h, (((1,), (1,)), ((), ())))

  return pl.pallas_call(
      body,
      grid=(NP // 1024,),
      in_specs=[pl.BlockSpec((2, 1024, 128), lambda i: (0, i, 0)),
                pl.BlockSpec((2, 8, 128), lambda i: (0, i, 0)),
                pl.BlockSpec((1, 128), lambda i: (0, 0)),
                pl.BlockSpec((128, 128), lambda i: (0, 0)),
                pl.BlockSpec((8, 128), lambda i: (0, 0))],
      out_specs=[pl.BlockSpec((1024, 128), lambda i: (i, 0)),
                 pl.BlockSpec((8, 1024), lambda i: (0, i))],
      out_shape=[jax.ShapeDtypeStruct((NP, 128), F32),
                 jax.ShapeDtypeStruct((8, NP), F32)],
  )(op, dp, bvec, W, A2)


def _tc_final(op, dp, bvec):
  def body(op_ref, dp_ref, b_ref, out_ref):
    out_ref[...] = _norm_block(op_ref, dp_ref) + b_ref[...]

  return pl.pallas_call(
      body,
      grid=(NP // 1024,),
      in_specs=[pl.BlockSpec((2, 1024, 128), lambda i: (0, i, 0)),
                pl.BlockSpec((2, 8, 128), lambda i: (0, i, 0)),
                pl.BlockSpec((1, 128), lambda i: (0, 0))],
      out_specs=pl.BlockSpec((1024, 128), lambda i: (i, 0)),
      out_shape=jax.ShapeDtypeStruct((NP, 128), F32),
  )(op, dp, bvec)


def kernel(x, edge_index, W1, a1_src, a1_dst, b1,
           W2, a2_src, a2_dst, b2, W3, a3_src, a3_dst, b3):
  ei = edge_index.astype(jnp.int32)
  src = ei[0]
  dst = ei[1]

  xp = jnp.pad(x, ((0, NP - N_NODES), (0, 0)))

  def a2rows(a_s, a_d, d):
    z = jnp.zeros((8, d), F32)
    return z.at[0, :a_s.shape[0]].set(a_s).at[1, :a_d.shape[0]].set(a_d)

  sc = _make_sc_agg()

  h1, st1 = _tc_first(xp, W1, a2rows(a1_src, a1_dst, 128))
  op1, dp1 = sc(h1, st1[0], st1[1], src, dst)

  h2, st2 = _tc_mid(op1, dp1, b1.reshape(1, 128),
                    W2, a2rows(a2_src, a2_dst, 128))
  op2, dp2 = sc(h2, st2[0], st2[1], src, dst)

  W3p = jnp.pad(W3, ((0, 0), (0, 128 - W3.shape[1])))
  h3, st3 = _tc_mid(op2, dp2, b2.reshape(1, 128),
                    W3p, a2rows(a3_src, a3_dst, 128))
  op3, dp3 = sc(h3, st3[0], st3[1], src, dst)

  b3p = jnp.pad(b3, (0, 128 - b3.shape[0])).reshape(1, 128)
  out = _tc_final(op3, dp3, b3p)
  return out[:N_NODES, :40]
